# Initial kernel scaffold; baseline (speedup 1.0000x reference)
import jax
import jax.numpy as jnp
from jax.experimental import pallas as pl


def kernel(x, edge_index, batch, W00, b00, W01, b01, W10, b10, W11, b11, W20, b20, W21, b21, W30, b30, W31, b31, Wout, bout):
    raise NotImplementedError("write your pallas kernel here")


# speedup vs baseline: 3.9929x; 3.9929x over previous
import functools

import jax
import jax.numpy as jnp
from jax import lax
from jax.experimental import pallas as pl
from jax.experimental.pallas import tpu as pltpu
import jax.experimental.pallas.tpu_sc as plsc

N = 10000
D = 128
E = 320000
G = 16
C = 2

NC = 2
NS = 16
NW = NC * NS
K = 128
CH = -(-E // (NW * K))
EP = NW * K * CH
NP = 10240
RPT = NP // NS

BLK = 512
NBLK = NP // BLK


def _sc_segment_body(h_hbm, src_hbm, dst_hbm, out_hbm, sidx, didx, rows, acc, sem):
    c = lax.axis_index("c")
    s = lax.axis_index("s")
    w = c * NS + s
    pltpu.sync_copy(h_hbm.at[pl.ds(s * RPT, RPT)], acc.at[pl.ds(s * RPT, RPT)])
    pltpu.sync_copy(src_hbm.at[w], sidx)
    pltpu.sync_copy(dst_hbm.at[w], didx)
    plsc.subcore_barrier()

    def chunk(j, carry):
        pltpu.async_copy(h_hbm.at[sidx.at[j]], rows, sem).wait()
        pltpu.sync_copy(rows, acc.at[didx.at[j]], add=True)
        return carry

    lax.fori_loop(0, CH, chunk, 0)
    plsc.subcore_barrier()
    pltpu.sync_copy(acc.at[pl.ds(s * RPT, RPT)],
                    out_hbm.at[c, pl.ds(s * RPT, RPT)])


@functools.cache
def _sc_segment():
    return pl.kernel(
        _sc_segment_body,
        out_type=jax.ShapeDtypeStruct((NC, NP, D), jnp.float32),
        mesh=plsc.VectorSubcoreMesh(core_axis_name="c", subcore_axis_name="s",
                                    num_cores=NC, num_subcores=NS),
        scratch_types=[
            pltpu.VMEM((CH, K), jnp.int32),
            pltpu.VMEM((CH, K), jnp.int32),
            pltpu.VMEM((K, D), jnp.float32),
            pltpu.VMEM_SHARED((NP, D), jnp.float32),
            pltpu.SemaphoreType.DMA,
        ],
    )


def _mlp_body(p0_ref, p1_ref, h_ref, w0_ref, b0_ref, w1_ref, b1_ref, o_ref):
    i = pl.program_id(0)
    t = p0_ref[...] + p1_ref[...] - h_ref[...]
    a = jnp.dot(t, w0_ref[...], preferred_element_type=jnp.float32) + b0_ref[...]
    a = jnp.maximum(a, 0.0)
    a = jnp.dot(a, w1_ref[...], preferred_element_type=jnp.float32) + b1_ref[...]
    a = jnp.maximum(a, 0.0)
    rows = i * BLK + lax.broadcasted_iota(jnp.int32, (BLK, 1), 0)
    o_ref[...] = jnp.where(rows < N, a, 0.0)


def _tc_mlp(p0, p1, h, w0, b0, w1, b1):
    row_spec = pl.BlockSpec((BLK, D), lambda i: (i, 0))
    full = lambda shape: pl.BlockSpec(shape, lambda i: (0, 0))
    return pl.pallas_call(
        _mlp_body,
        grid=(NBLK,),
        in_specs=[row_spec, row_spec, row_spec,
                  full((D, D)), full((1, D)), full((D, D)), full((1, D))],
        out_specs=row_spec,
        out_shape=jax.ShapeDtypeStruct((NP, D), jnp.float32),
    )(p0, p1, h, w0, b0, w1, b1)


def _final_body(p0_ref, p1_ref, h_ref, batch_ref, w0_ref, b0_ref, w1_ref,
                b1_ref, wout_ref, bout_ref, o_ref, pooled):
    i = pl.program_id(0)
    t = p0_ref[...] + p1_ref[...] - h_ref[...]
    a = jnp.dot(t, w0_ref[...], preferred_element_type=jnp.float32) + b0_ref[...]
    a = jnp.maximum(a, 0.0)
    a = jnp.dot(a, w1_ref[...], preferred_element_type=jnp.float32) + b1_ref[...]
    a = jnp.maximum(a, 0.0)

    @pl.when(i == 0)
    def _():
        pooled[...] = jnp.full((G, D), -jnp.inf, dtype=jnp.float32)

    b = batch_ref[...]
    for g in range(G):
        v = jnp.where(b == g, a, -jnp.inf).max(axis=0)
        pooled[g, :] = jnp.maximum(pooled[g, :], v)

    @pl.when(i == NBLK - 1)
    def _():
        o_ref[...] = (
            jnp.dot(pooled[...], wout_ref[...], preferred_element_type=jnp.float32)
            + bout_ref[...])


def _tc_final(p0, p1, h, batch2d, w0, b0, w1, b1, wout, bout):
    row_spec = pl.BlockSpec((BLK, D), lambda i: (i, 0))
    full = lambda shape: pl.BlockSpec(shape, lambda i: (0, 0))
    return pl.pallas_call(
        _final_body,
        grid=(NBLK,),
        in_specs=[row_spec, row_spec, row_spec,
                  pl.BlockSpec((BLK, 1), lambda i: (i, 0)),
                  full((D, D)), full((1, D)), full((D, D)), full((1, D)),
                  full((D, C)), full((1, C))],
        out_specs=full((G, C)),
        out_shape=jax.ShapeDtypeStruct((G, C), jnp.float32),
        scratch_shapes=[pltpu.VMEM((G, D), jnp.float32)],
    )(p0, p1, h, batch2d, w0, b0, w1, b1, wout, bout)


def kernel(x, edge_index, batch, W00, b00, W01, b01, W10, b10, W11, b11,
           W20, b20, W21, b21, W30, b30, W31, b31, Wout, bout):
    xp = jnp.zeros((NP, D), jnp.float32).at[:N].set(x)
    npad = EP - E
    srcp = jnp.concatenate([edge_index[0], jnp.full((npad,), N, jnp.int32)])
    dstp = jnp.concatenate(
        [edge_index[1], N + (jnp.arange(npad, dtype=jnp.int32) % (NP - N))])
    srcp = srcp.reshape(NW, CH, K)
    dstp = dstp.reshape(NW, CH, K)
    batch2d = jnp.full((NP, 1), G, jnp.int32).at[:N, 0].set(batch)

    ws = [(W00, b00.reshape(1, D), W01, b01.reshape(1, D)),
          (W10, b10.reshape(1, D), W11, b11.reshape(1, D)),
          (W20, b20.reshape(1, D), W21, b21.reshape(1, D)),
          (W30, b30.reshape(1, D), W31, b31.reshape(1, D))]

    seg = _sc_segment()
    h = xp
    for l in range(3):
        p = seg(h, srcp, dstp)
        h = _tc_mlp(p[0], p[1], h, *ws[l])
    p = seg(h, srcp, dstp)
    return _tc_final(p[0], p[1], h, batch2d, *ws[3],
                     Wout, bout.reshape(1, C))
